# baseline (speedup 1.0000x reference)
import jax
import jax.numpy as jnp
from jax.experimental import pallas as pl
from jax.experimental.pallas import tpu as pltpu

NE = 8192
D = 256
NT = 8192
TT = 256
COMMIT_W = 0.25


def _vq_body(x_ref, w_ref, sx_ref, se_ref, enc_ref, q_ref, loss_ref):
    t = pl.program_id(0)

    xt = x_ref[...]
    w = w_ref[...]
    mm = jnp.dot(xt, w.T, preferred_element_type=jnp.float32)
    d = (sx_ref[...] + se_ref[...]) - 2.0 * mm

    dmin = jnp.min(d, axis=1, keepdims=True)
    iota = jax.lax.broadcasted_iota(jnp.int32, (TT, NE), 1)
    idx = jnp.min(jnp.where(d == dmin, iota, NE), axis=1, keepdims=True)

    enc = (iota == idx).astype(jnp.float32)
    enc_ref[...] = enc

    q_ref[...] = jnp.dot(enc.astype(jnp.bfloat16), w.astype(jnp.bfloat16),
                         preferred_element_type=jnp.float32)

    @pl.when(t == 0)
    def _():
        loss_ref[...] = jnp.zeros((1, 1), jnp.float32)
    loss_ref[...] += jnp.sum(dmin).reshape(1, 1)


def kernel(x, embedding_weight):
    xp = jnp.transpose(x, (0, 2, 3, 1))
    flat_x = xp.reshape(NT, D)
    sx = jnp.sum(flat_x ** 2, axis=1, keepdims=True)
    se = jnp.sum(embedding_weight ** 2, axis=1)[None, :]

    grid = (NT // TT,)
    enc, qf, loss_acc = pl.pallas_call(
        _vq_body,
        grid=grid,
        in_specs=[
            pl.BlockSpec((TT, D), lambda t: (t, 0)),
            pl.BlockSpec((NE, D), lambda t: (0, 0)),
            pl.BlockSpec((TT, 1), lambda t: (t, 0)),
            pl.BlockSpec((1, NE), lambda t: (0, 0)),
        ],
        out_specs=[
            pl.BlockSpec((TT, NE), lambda t: (t, 0)),
            pl.BlockSpec((TT, D), lambda t: (t, 0)),
            pl.BlockSpec((1, 1), lambda t: (0, 0)),
        ],
        out_shape=[
            jax.ShapeDtypeStruct((NT, NE), jnp.float32),
            jax.ShapeDtypeStruct((NT, D), jnp.float32),
            jax.ShapeDtypeStruct((1, 1), jnp.float32),
        ],
    )(flat_x, embedding_weight, sx, se)

    loss = COMMIT_W * (loss_acc[0, 0] / (NT * D))
    quantized = jnp.transpose(qf.reshape(8, 32, 32, D), (0, 3, 1, 2))
    return (loss, quantized, enc)


# speedup vs baseline: 1.4638x; 1.4638x over previous
import functools

import jax
import jax.numpy as jnp
from jax import lax
from jax.experimental import pallas as pl
from jax.experimental.pallas import tpu as pltpu
from jax.experimental.pallas import tpu_sc as plsc

NE = 8192
D = 256
NT = 8192
TT = 256
COMMIT_W = 0.25

_NW = 32
_BPW = NT // _NW


def _vq_body(x2_ref, w_ref, sx_ref, se_ref, enc_ref, idx_ref, loss_ref):
    t = pl.program_id(0)

    mm2 = jnp.dot(x2_ref[...], w_ref[...].T,
                  preferred_element_type=jnp.float32)
    d = (sx_ref[...] + se_ref[...]) - mm2

    dmin = jnp.min(d, axis=1, keepdims=True)
    iota_f = jax.lax.broadcasted_iota(jnp.int32, (TT, NE), 1).astype(jnp.float32)
    idxf = jnp.min(jnp.where(d == dmin, iota_f, jnp.inf), axis=1,
                   keepdims=True)
    enc_ref[...] = (iota_f == idxf).astype(jnp.float32)
    idx_ref[...] = idxf.astype(jnp.int32)

    @pl.when(t == 0)
    def _():
        loss_ref[...] = jnp.zeros((1, 1), jnp.float32)
    loss_ref[...] += jnp.sum(dmin).reshape(1, 1)


@functools.partial(
    pl.kernel,
    mesh=plsc.VectorSubcoreMesh(core_axis_name="c", subcore_axis_name="s"),
    out_type=jax.ShapeDtypeStruct((NT, D), jnp.float32),
    scratch_types=[
        pltpu.VMEM((_BPW,), jnp.int32),
        pltpu.VMEM((_BPW, D), jnp.float32),
        pltpu.SemaphoreType.DMA,
    ],
)
def _sc_gather(table_hbm, idx_hbm, out_hbm, idx_v, rows_v, sem):
    wid = lax.axis_index("s") * 2 + lax.axis_index("c")
    base = wid * _BPW
    pltpu.sync_copy(idx_hbm.at[pl.ds(base, _BPW)], idx_v)
    pltpu.async_copy(table_hbm.at[idx_v], rows_v, sem).wait()
    pltpu.sync_copy(rows_v, out_hbm.at[pl.ds(base, _BPW)])


def kernel(x, embedding_weight):
    xp = jnp.transpose(x, (0, 2, 3, 1))
    flat_x = xp.reshape(NT, D)
    sx = jnp.sum(flat_x ** 2, axis=1, keepdims=True)
    se = jnp.sum(embedding_weight ** 2, axis=1)[None, :]

    grid = (NT // TT,)
    enc, idx, loss_acc = pl.pallas_call(
        _vq_body,
        grid=grid,
        in_specs=[
            pl.BlockSpec((TT, D), lambda t: (t, 0)),
            pl.BlockSpec((NE, D), lambda t: (0, 0)),
            pl.BlockSpec((TT, 1), lambda t: (t, 0)),
            pl.BlockSpec((1, NE), lambda t: (0, 0)),
        ],
        out_specs=[
            pl.BlockSpec((TT, NE), lambda t: (t, 0)),
            pl.BlockSpec((TT, 1), lambda t: (t, 0)),
            pl.BlockSpec((1, 1), lambda t: (0, 0)),
        ],
        out_shape=[
            jax.ShapeDtypeStruct((NT, NE), jnp.float32),
            jax.ShapeDtypeStruct((NT, 1), jnp.int32),
            jax.ShapeDtypeStruct((1, 1), jnp.float32),
        ],
    )(flat_x * 2.0, embedding_weight, sx, se)

    qf = _sc_gather(embedding_weight, idx.reshape(NT))

    loss = COMMIT_W * (loss_acc[0, 0] / (NT * D))
    quantized = jnp.transpose(qf.reshape(8, 32, 32, D), (0, 3, 1, 2))
    return (loss, quantized, enc)
